# baseline (speedup 1.0000x reference)
import functools

import jax
import jax.numpy as jnp
from jax import lax
from jax.experimental import pallas as pl
from jax.experimental.pallas import tpu as pltpu
from jax.experimental.pallas import tpu_sc as plsc

N = 50000
E = 800000
D_IN = 27
D_E = 8
HID = 64
H = 4
C = 16
L = 3

NHALF = N // 2
ACC_ROWS = 25024
DUMP_ROW = 25008
ACCW = 80
KE = 48
TILES = 16
E_PAD = 800256
EPT = E_PAD // TILES
WTILE = ACC_ROWS // TILES
ROWS_OUT = 1560

_BN = 1000


def _ln(x, g, b, eps=1e-5):
    mu = jnp.mean(x, axis=-1, keepdims=True)
    var = jnp.mean((x - mu) ** 2, axis=-1, keepdims=True)
    return (x - mu) / jnp.sqrt(var + eps) * g + b


def _gelu(x):
    return x * 0.5 * (1.0 + lax.erf(x / jnp.sqrt(2.0).astype(x.dtype)))


def _inproj_body(x_ref, w_ref, b_ref, g_ref, b2_ref, out_ref):
    h = jnp.dot(x_ref[...], w_ref[...], preferred_element_type=jnp.float32)
    out_ref[...] = _gelu(_ln(h + b_ref[...], g_ref[...], b2_ref[...]))


def _tables_body(h_ref, w_ref, as_ref, ad_ref, tsrc_ref, tdst_ref):
    xl = jnp.dot(h_ref[...], w_ref[...], preferred_element_type=jnp.float32)
    asrc = jnp.dot(xl, as_ref[...], preferred_element_type=jnp.float32)
    adst = jnp.dot(xl, ad_ref[...], preferred_element_type=jnp.float32)
    pad = jnp.zeros((xl.shape[0], C - H), jnp.float32)
    tsrc_ref[...] = jnp.concatenate([xl, asrc, pad], axis=-1)
    tdst_ref[...] = jnp.concatenate([adst, pad], axis=-1)


def _ae_body(ea_ref, v0_ref, v1_ref, v2_ref, o0_ref, o1_ref, o2_ref):
    ea = ea_ref[...]
    pad = jnp.zeros((ea.shape[0], C - H), jnp.float32)
    for v_ref, o_ref in ((v0_ref, o0_ref), (v1_ref, o1_ref), (v2_ref, o2_ref)):
        a = jnp.dot(ea, v_ref[...], preferred_element_type=jnp.float32)
        o_ref[...] = jnp.concatenate([a, pad], axis=-1)


def _post_body(acc_ref, h_ref, gb_ref, g_ref, b_ref, out_ref):
    a = acc_ref[...]
    num = a[:, :HID]
    den = a[:, HID:HID + H]
    agg = num.reshape(-1, H, C) / (den[:, :, None] + 1e-16)
    hh = agg.reshape(-1, HID) + gb_ref[...] + h_ref[...]
    out_ref[...] = _gelu(_ln(hh, g_ref[...], b_ref[...]))


def _heads_body(h_ref, f1w_ref, f1b_ref, f2w_ref, f2b_ref, lng_ref, lnb_ref,
                pw_ref, pb_ref, cf1w_ref, cf1b_ref, cf2w_ref, cf2b_ref,
                clng_ref, clnb_ref, cpw_ref, cpb_ref, out_ref):
    hv = h_ref[...]
    scores = []
    for i in range(4):
        t = _gelu(jnp.dot(hv, f1w_ref[i], preferred_element_type=jnp.float32)
                  + f1b_ref[i][None, :])
        t = jnp.dot(t, f2w_ref[i], preferred_element_type=jnp.float32) + f2b_ref[i][None, :]
        p = jnp.dot(hv, pw_ref[i], preferred_element_type=jnp.float32) + pb_ref[i][None, :]
        z = _ln(t + p, lng_ref[i][None, :], lnb_ref[i][None, :])
        scores.append(jax.nn.sigmoid(z))
    comp_in = jnp.concatenate([hv] + scores, axis=-1)
    t = _gelu(jnp.dot(comp_in, cf1w_ref[...], preferred_element_type=jnp.float32)
              + cf1b_ref[...])
    t = jnp.dot(t, cf2w_ref[...], preferred_element_type=jnp.float32) + cf2b_ref[...]
    p = jnp.dot(comp_in, cpw_ref[...], preferred_element_type=jnp.float32) + cpb_ref[...]
    comp = jax.nn.sigmoid(_ln(t + p, clng_ref[...], clnb_ref[...]))
    out_ref[...] = jnp.concatenate([comp] + scores, axis=-1)


def _full(shape):
    nd = len(shape)
    return pl.BlockSpec(shape, lambda i, _nd=nd: (0,) * _nd)


def _rows(width):
    return pl.BlockSpec((_BN, width), lambda i: (i, 0))


_SPLAT_DN = lax.GatherDimensionNumbers(
    offset_dims=(), collapsed_slice_dims=(0,), start_index_map=(0,))


def _lane_splat(v, h):
    idx = jnp.full((16, 1), h, jnp.int32)
    return lax.gather(v, idx, _SPLAT_DN, (1,),
                      mode=lax.GatherScatterMode.PROMISE_IN_BOUNDS)


def _sc_agg_body(zeros_hbm, tsrc_hbm, tdst_hbm, sd_hbm,
                 out_hbm, sdv, dloc, rows_src, rows_dst,
                 acc, sem1, sem2):
    c = lax.axis_index("c")
    s = lax.axis_index("s")

    pltpu.sync_copy(zeros_hbm, rows_src)

    def zfull(t, carry):
        pltpu.sync_copy(rows_src, acc.at[pl.ds(s * WTILE + t * KE, KE)])
        return carry

    lax.fori_loop(0, WTILE // KE, zfull, 0)
    pltpu.sync_copy(rows_src.at[pl.ds(0, WTILE % KE)],
                    acc.at[pl.ds(s * WTILE + (WTILE // KE) * KE, WTILE % KE)])
    plsc.subcore_barrier()

    base0 = s * EPT
    off = c * NHALF

    def edge(e, carry2):
        ae_c = plsc.bitcast(sdv[pl.ds(2 * KE + e * C, C)], jnp.float32)
        ex = rows_src[e, pl.ds(HID, C)] + rows_dst[e, pl.ds(0, C)] + ae_c
        ex = jnp.where(ex >= 0.0, ex, 0.2 * ex)
        ex = jnp.exp(ex)
        for hh in range(H):
            m = _lane_splat(ex, hh)
            rows_src[e, pl.ds(hh * C, C)] = rows_src[e, pl.ds(hh * C, C)] * m
        rows_src[e, pl.ds(HID, C)] = ex
        return carry2

    chunk0 = base0 // KE

    def chunk(i, carry):
        pltpu.sync_copy(sd_hbm.at[chunk0 + i], sdv)
        cp1 = pltpu.async_copy(tsrc_hbm.at[sdv.at[pl.ds(0, KE)]],
                               rows_src, sem1)
        cp2 = pltpu.async_copy(tdst_hbm.at[sdv.at[pl.ds(KE, KE)]],
                               rows_dst, sem2)
        for q in range(KE // 16):
            dd = sdv[pl.ds(KE + q * 16, 16)] - off
            ok = (dd >= 0) & (dd < NHALF)
            dloc[pl.ds(q * 16, 16)] = jnp.where(ok, dd, DUMP_ROW)
        cp1.wait()
        cp2.wait()
        lax.fori_loop(0, KE, edge, 0, unroll=4)
        pltpu.sync_copy(rows_src, acc.at[dloc], add=True)
        return carry

    lax.fori_loop(0, EPT // KE, chunk, 0)
    plsc.subcore_barrier()

    row_lo = s * ROWS_OUT
    pltpu.sync_copy(acc.at[pl.ds(row_lo, ROWS_OUT)],
                    out_hbm.at[pl.ds(off + row_lo, ROWS_OUT)])

    @pl.when(s == 0)
    def _():
        pltpu.sync_copy(acc.at[pl.ds(TILES * ROWS_OUT, NHALF - TILES * ROWS_OUT)],
                        out_hbm.at[pl.ds(off + TILES * ROWS_OUT,
                                         NHALF - TILES * ROWS_OUT)])


@functools.cache
def _make_sc_agg():
    return pl.kernel(
        _sc_agg_body,
        out_type=jax.ShapeDtypeStruct((N, ACCW), jnp.float32),
        mesh=plsc.VectorSubcoreMesh(core_axis_name="c", subcore_axis_name="s",
                                    num_cores=2, num_subcores=TILES),
        compiler_params=pltpu.CompilerParams(use_tc_tiling_on_sc=False,
                                             needs_layout_passes=False),
        scratch_types=[
            pltpu.VMEM(((2 + C) * KE,), jnp.int32),
            pltpu.VMEM((KE,), jnp.int32),
            pltpu.VMEM((KE, ACCW), jnp.float32),
            pltpu.VMEM((KE, C), jnp.float32),
            pltpu.VMEM_SHARED((ACC_ROWS, ACCW), jnp.float32),
            pltpu.SemaphoreType.DMA,
            pltpu.SemaphoreType.DMA,
        ],
    )


def kernel(x, edge_index, edge_attr, in_W, in_b, in_ln_g, in_ln_b, gat_W,
           gat_att_src, gat_att_dst, gat_att_edge, gat_edge_W, gat_b, ln_g,
           ln_b, head_fc1_W, head_fc1_b, head_fc2_W, head_fc2_b, head_ln_g,
           head_ln_b, head_proj_W, head_proj_b, comp_fc1_W, comp_fc1_b,
           comp_fc2_W, comp_fc2_b, comp_ln_g, comp_ln_b, comp_proj_W,
           comp_proj_b):
    f32 = jnp.float32
    npad = E_PAD - E
    src = jnp.concatenate([edge_index[0].astype(jnp.int32),
                           jnp.zeros((npad,), jnp.int32)])
    dst = jnp.concatenate([edge_index[1].astype(jnp.int32),
                           jnp.full((npad,), N, jnp.int32)])
    grid_n = N // _BN

    eye = jnp.eye(H, dtype=f32)
    As = [(gat_att_src[l][:, :, None] * eye[:, None, :]).reshape(HID, H)
          for l in range(L)]
    Ad = [(gat_att_dst[l][:, :, None] * eye[:, None, :]).reshape(HID, H)
          for l in range(L)]
    Ve = [jnp.einsum("dhc,hc->dh", gat_edge_W[l].reshape(D_E, H, C),
                     gat_att_edge[l]) for l in range(L)]

    h = pl.pallas_call(
        _inproj_body,
        grid=(grid_n,),
        in_specs=[_rows(D_IN), _full((D_IN, HID)), _full((1, HID)),
                  _full((1, HID)), _full((1, HID))],
        out_specs=_rows(HID),
        out_shape=jax.ShapeDtypeStruct((N, HID), f32),
    )(x, in_W, in_b.reshape(1, HID), in_ln_g.reshape(1, HID),
      in_ln_b.reshape(1, HID))

    grid_e = E // 2000
    espec = pl.BlockSpec((2000, D_E), lambda i: (i, 0))
    oespec = pl.BlockSpec((2000, C), lambda i: (i, 0))
    ae = pl.pallas_call(
        _ae_body,
        grid=(grid_e,),
        in_specs=[espec, _full((D_E, H)), _full((D_E, H)), _full((D_E, H))],
        out_specs=[oespec, oespec, oespec],
        out_shape=[jax.ShapeDtypeStruct((E, C), f32)] * 3,
    )(edge_attr, Ve[0], Ve[1], Ve[2])
    aepad = jnp.zeros((npad, C), f32)
    src2 = src.reshape(-1, KE)
    dst2 = dst.reshape(-1, KE)
    sd = [jnp.concatenate([
        src2, dst2,
        lax.bitcast_convert_type(jnp.concatenate([a, aepad]),
                                 jnp.int32).reshape(-1, C * KE)], axis=1)
        for a in ae]

    zeros_hbm = jnp.zeros((KE, ACCW), f32)

    for l in range(L):
        tsrc, tdst = pl.pallas_call(
            _tables_body,
            grid=(grid_n,),
            in_specs=[_rows(HID), _full((HID, HID)), _full((HID, H)),
                      _full((HID, H))],
            out_specs=[_rows(ACCW), _rows(C)],
            out_shape=[jax.ShapeDtypeStruct((N, ACCW), f32),
                       jax.ShapeDtypeStruct((N, C), f32)],
        )(h, gat_W[l], As[l], Ad[l])

        acc = _make_sc_agg()(zeros_hbm, tsrc, tdst, sd[l])

        h = pl.pallas_call(
            _post_body,
            grid=(grid_n,),
            in_specs=[_rows(ACCW), _rows(HID), _full((1, HID)),
                      _full((1, HID)), _full((1, HID))],
            out_specs=_rows(HID),
            out_shape=jax.ShapeDtypeStruct((N, HID), f32),
        )(acc, h, gat_b[l].reshape(1, HID), ln_g[l].reshape(1, HID),
          ln_b[l].reshape(1, HID))

    out = pl.pallas_call(
        _heads_body,
        grid=(grid_n,),
        in_specs=[_rows(HID), _full((4, HID, HID // 2)), _full((4, HID // 2)),
                  _full((4, HID // 2, 1)), _full((4, 1)), _full((4, 1)),
                  _full((4, 1)), _full((4, HID, 1)), _full((4, 1)),
                  _full((HID + 4, HID // 2)), _full((1, HID // 2)),
                  _full((HID // 2, 1)), _full((1, 1)), _full((1, 1)),
                  _full((1, 1)), _full((HID + 4, 1)), _full((1, 1))],
        out_specs=_rows(5),
        out_shape=jax.ShapeDtypeStruct((N, 5), f32),
    )(h, head_fc1_W, head_fc1_b, head_fc2_W, head_fc2_b, head_ln_g,
      head_ln_b, head_proj_W, head_proj_b, comp_fc1_W,
      comp_fc1_b.reshape(1, HID // 2), comp_fc2_W,
      comp_fc2_b.reshape(1, 1), comp_ln_g.reshape(1, 1),
      comp_ln_b.reshape(1, 1), comp_proj_W, comp_proj_b.reshape(1, 1))
    return out


# speedup vs baseline: 1.1270x; 1.1270x over previous
import functools

import jax
import jax.numpy as jnp
from jax import lax
from jax.experimental import pallas as pl
from jax.experimental.pallas import tpu as pltpu
from jax.experimental.pallas import tpu_sc as plsc

N = 50000
E = 800000
D_IN = 27
D_E = 8
HID = 64
H = 4
C = 16
L = 3

NHALF = N // 2
ACC_ROWS = 25024
DUMP_ROW = 25008
ACCW = 80
KE = 48
TILES = 16
E_PAD = 800256
EPT = E_PAD // TILES
WTILE = ACC_ROWS // TILES
ROWS_OUT = 1560

_BN = 1000


def _ln(x, g, b, eps=1e-5):
    mu = jnp.mean(x, axis=-1, keepdims=True)
    var = jnp.mean((x - mu) ** 2, axis=-1, keepdims=True)
    return (x - mu) / jnp.sqrt(var + eps) * g + b


def _gelu(x):
    return x * 0.5 * (1.0 + lax.erf(x / jnp.sqrt(2.0).astype(x.dtype)))


def _inproj_body(x_ref, w_ref, b_ref, g_ref, b2_ref, out_ref):
    h = jnp.dot(x_ref[...], w_ref[...], preferred_element_type=jnp.float32)
    out_ref[...] = _gelu(_ln(h + b_ref[...], g_ref[...], b2_ref[...]))


def _tables_body(h_ref, w_ref, as_ref, ad_ref, tsrc_ref, tdst_ref):
    xl = jnp.dot(h_ref[...], w_ref[...], preferred_element_type=jnp.float32)
    asrc = jnp.dot(xl, as_ref[...], preferred_element_type=jnp.float32)
    adst = jnp.dot(xl, ad_ref[...], preferred_element_type=jnp.float32)
    pad = jnp.zeros((xl.shape[0], C - H), jnp.float32)
    tsrc_ref[...] = jnp.concatenate([xl, asrc, pad], axis=-1)
    tdst_ref[...] = jnp.concatenate([adst, pad], axis=-1)


def _ae_body(ea_ref, v0_ref, v1_ref, v2_ref, o0_ref, o1_ref, o2_ref):
    ea = ea_ref[...]
    pad = jnp.zeros((ea.shape[0], C - H), jnp.float32)
    for v_ref, o_ref in ((v0_ref, o0_ref), (v1_ref, o1_ref), (v2_ref, o2_ref)):
        a = jnp.dot(ea, v_ref[...], preferred_element_type=jnp.float32)
        o_ref[...] = jnp.concatenate([a, pad], axis=-1)


def _post_body(acc_ref, h_ref, gb_ref, g_ref, b_ref, out_ref):
    a = acc_ref[...]
    num = a[:, :HID]
    den = a[:, HID:HID + H]
    agg = num.reshape(-1, H, C) / (den[:, :, None] + 1e-16)
    hh = agg.reshape(-1, HID) + gb_ref[...] + h_ref[...]
    out_ref[...] = _gelu(_ln(hh, g_ref[...], b_ref[...]))


def _heads_body(h_ref, f1w_ref, f1b_ref, f2w_ref, f2b_ref, lng_ref, lnb_ref,
                pw_ref, pb_ref, cf1w_ref, cf1b_ref, cf2w_ref, cf2b_ref,
                clng_ref, clnb_ref, cpw_ref, cpb_ref, out_ref):
    hv = h_ref[...]
    scores = []
    for i in range(4):
        t = _gelu(jnp.dot(hv, f1w_ref[i], preferred_element_type=jnp.float32)
                  + f1b_ref[i][None, :])
        t = jnp.dot(t, f2w_ref[i], preferred_element_type=jnp.float32) + f2b_ref[i][None, :]
        p = jnp.dot(hv, pw_ref[i], preferred_element_type=jnp.float32) + pb_ref[i][None, :]
        z = _ln(t + p, lng_ref[i][None, :], lnb_ref[i][None, :])
        scores.append(jax.nn.sigmoid(z))
    comp_in = jnp.concatenate([hv] + scores, axis=-1)
    t = _gelu(jnp.dot(comp_in, cf1w_ref[...], preferred_element_type=jnp.float32)
              + cf1b_ref[...])
    t = jnp.dot(t, cf2w_ref[...], preferred_element_type=jnp.float32) + cf2b_ref[...]
    p = jnp.dot(comp_in, cpw_ref[...], preferred_element_type=jnp.float32) + cpb_ref[...]
    comp = jax.nn.sigmoid(_ln(t + p, clng_ref[...], clnb_ref[...]))
    out_ref[...] = jnp.concatenate([comp] + scores, axis=-1)


def _full(shape):
    nd = len(shape)
    return pl.BlockSpec(shape, lambda i, _nd=nd: (0,) * _nd)


def _rows(width):
    return pl.BlockSpec((_BN, width), lambda i: (i, 0))


_SPLAT_DN = lax.GatherDimensionNumbers(
    offset_dims=(), collapsed_slice_dims=(0,), start_index_map=(0,))


def _lane_splat(v, h):
    idx = jnp.full((16, 1), h, jnp.int32)
    return lax.gather(v, idx, _SPLAT_DN, (1,),
                      mode=lax.GatherScatterMode.PROMISE_IN_BOUNDS)


def _sc_agg_body(zeros_hbm, tsrc_hbm, tdst_hbm, ae_hbm, sd_hbm,
                 out_hbm, sdv, dloc, rows_src, rows_dst, rows_ae,
                 acc, sem1, sem2):
    c = lax.axis_index("c")
    s = lax.axis_index("s")

    pltpu.sync_copy(zeros_hbm, rows_src)

    def zfull(t, carry):
        pltpu.sync_copy(rows_src, acc.at[pl.ds(s * WTILE + t * KE, KE)])
        return carry

    lax.fori_loop(0, WTILE // KE, zfull, 0)
    pltpu.sync_copy(rows_src.at[pl.ds(0, WTILE % KE)],
                    acc.at[pl.ds(s * WTILE + (WTILE // KE) * KE, WTILE % KE)])
    plsc.subcore_barrier()

    base0 = s * EPT
    off = c * NHALF

    def edge(e, carry2):
        ex = (rows_src[e, pl.ds(HID, C)] + rows_dst[e, pl.ds(0, C)]
              + rows_ae[e, pl.ds(0, C)])
        ex = jnp.where(ex >= 0.0, ex, 0.2 * ex)
        ex = jnp.exp(ex)
        for hh in range(H):
            m = _lane_splat(ex, hh)
            rows_src[e, pl.ds(hh * C, C)] = rows_src[e, pl.ds(hh * C, C)] * m
        rows_src[e, pl.ds(HID, C)] = ex
        return carry2

    chunk0 = base0 // KE

    def chunk(i, carry):
        pltpu.sync_copy(sd_hbm.at[chunk0 + i], sdv)
        cp1 = pltpu.async_copy(tsrc_hbm.at[sdv.at[pl.ds(0, KE)]],
                               rows_src, sem1)
        cp2 = pltpu.async_copy(tdst_hbm.at[sdv.at[pl.ds(KE, KE)]],
                               rows_dst, sem2)
        pltpu.sync_copy(ae_hbm.at[pl.ds(base0 + i * KE, KE)], rows_ae)
        for q in range(KE // 16):
            dd = sdv[pl.ds(KE + q * 16, 16)] - off
            ok = (dd >= 0) & (dd < NHALF)
            dloc[pl.ds(q * 16, 16)] = jnp.where(ok, dd, DUMP_ROW)
        cp1.wait()
        cp2.wait()
        lax.fori_loop(0, KE, edge, 0, unroll=4)
        pltpu.sync_copy(rows_src, acc.at[dloc], add=True)
        return carry

    lax.fori_loop(0, EPT // KE, chunk, 0)
    plsc.subcore_barrier()

    row_lo = s * ROWS_OUT
    pltpu.sync_copy(acc.at[pl.ds(row_lo, ROWS_OUT)],
                    out_hbm.at[pl.ds(off + row_lo, ROWS_OUT)])

    @pl.when(s == 0)
    def _():
        pltpu.sync_copy(acc.at[pl.ds(TILES * ROWS_OUT, NHALF - TILES * ROWS_OUT)],
                        out_hbm.at[pl.ds(off + TILES * ROWS_OUT,
                                         NHALF - TILES * ROWS_OUT)])


@functools.cache
def _make_sc_agg():
    return pl.kernel(
        _sc_agg_body,
        out_type=jax.ShapeDtypeStruct((N, ACCW), jnp.float32),
        mesh=plsc.VectorSubcoreMesh(core_axis_name="c", subcore_axis_name="s",
                                    num_cores=2, num_subcores=TILES),
        compiler_params=pltpu.CompilerParams(use_tc_tiling_on_sc=False),
        scratch_types=[
            pltpu.VMEM((2 * KE,), jnp.int32),
            pltpu.VMEM((KE,), jnp.int32),
            pltpu.VMEM((KE, ACCW), jnp.float32),
            pltpu.VMEM((KE, C), jnp.float32),
            pltpu.VMEM((KE, C), jnp.float32),
            pltpu.VMEM_SHARED((ACC_ROWS, ACCW), jnp.float32),
            pltpu.SemaphoreType.DMA,
            pltpu.SemaphoreType.DMA,
        ],
    )


def kernel(x, edge_index, edge_attr, in_W, in_b, in_ln_g, in_ln_b, gat_W,
           gat_att_src, gat_att_dst, gat_att_edge, gat_edge_W, gat_b, ln_g,
           ln_b, head_fc1_W, head_fc1_b, head_fc2_W, head_fc2_b, head_ln_g,
           head_ln_b, head_proj_W, head_proj_b, comp_fc1_W, comp_fc1_b,
           comp_fc2_W, comp_fc2_b, comp_ln_g, comp_ln_b, comp_proj_W,
           comp_proj_b):
    f32 = jnp.float32
    npad = E_PAD - E
    src = jnp.concatenate([edge_index[0].astype(jnp.int32),
                           jnp.zeros((npad,), jnp.int32)])
    dst = jnp.concatenate([edge_index[1].astype(jnp.int32),
                           jnp.full((npad,), N, jnp.int32)])
    grid_n = N // _BN

    eye = jnp.eye(H, dtype=f32)
    As = [(gat_att_src[l][:, :, None] * eye[:, None, :]).reshape(HID, H)
          for l in range(L)]
    Ad = [(gat_att_dst[l][:, :, None] * eye[:, None, :]).reshape(HID, H)
          for l in range(L)]
    Ve = [jnp.einsum("dhc,hc->dh", gat_edge_W[l].reshape(D_E, H, C),
                     gat_att_edge[l]) for l in range(L)]

    h = pl.pallas_call(
        _inproj_body,
        grid=(grid_n,),
        in_specs=[_rows(D_IN), _full((D_IN, HID)), _full((1, HID)),
                  _full((1, HID)), _full((1, HID))],
        out_specs=_rows(HID),
        out_shape=jax.ShapeDtypeStruct((N, HID), f32),
    )(x, in_W, in_b.reshape(1, HID), in_ln_g.reshape(1, HID),
      in_ln_b.reshape(1, HID))

    grid_e = E // 2000
    espec = pl.BlockSpec((2000, D_E), lambda i: (i, 0))
    oespec = pl.BlockSpec((2000, C), lambda i: (i, 0))
    ae = pl.pallas_call(
        _ae_body,
        grid=(grid_e,),
        in_specs=[espec, _full((D_E, H)), _full((D_E, H)), _full((D_E, H))],
        out_specs=[oespec, oespec, oespec],
        out_shape=[jax.ShapeDtypeStruct((E, C), f32)] * 3,
    )(edge_attr, Ve[0], Ve[1], Ve[2])
    aepad = jnp.zeros((npad, C), f32)
    ae = [jnp.concatenate([a, aepad]) for a in ae]
    sd = jnp.concatenate([src.reshape(-1, KE), dst.reshape(-1, KE)], axis=1)

    zeros_hbm = jnp.zeros((KE, ACCW), f32)

    for l in range(L):
        tsrc, tdst = pl.pallas_call(
            _tables_body,
            grid=(grid_n,),
            in_specs=[_rows(HID), _full((HID, HID)), _full((HID, H)),
                      _full((HID, H))],
            out_specs=[_rows(ACCW), _rows(C)],
            out_shape=[jax.ShapeDtypeStruct((N, ACCW), f32),
                       jax.ShapeDtypeStruct((N, C), f32)],
        )(h, gat_W[l], As[l], Ad[l])

        acc = _make_sc_agg()(zeros_hbm, tsrc, tdst, ae[l], sd)

        h = pl.pallas_call(
            _post_body,
            grid=(grid_n,),
            in_specs=[_rows(ACCW), _rows(HID), _full((1, HID)),
                      _full((1, HID)), _full((1, HID))],
            out_specs=_rows(HID),
            out_shape=jax.ShapeDtypeStruct((N, HID), f32),
        )(acc, h, gat_b[l].reshape(1, HID), ln_g[l].reshape(1, HID),
          ln_b[l].reshape(1, HID))

    out = pl.pallas_call(
        _heads_body,
        grid=(grid_n,),
        in_specs=[_rows(HID), _full((4, HID, HID // 2)), _full((4, HID // 2)),
                  _full((4, HID // 2, 1)), _full((4, 1)), _full((4, 1)),
                  _full((4, 1)), _full((4, HID, 1)), _full((4, 1)),
                  _full((HID + 4, HID // 2)), _full((1, HID // 2)),
                  _full((HID // 2, 1)), _full((1, 1)), _full((1, 1)),
                  _full((1, 1)), _full((HID + 4, 1)), _full((1, 1))],
        out_specs=_rows(5),
        out_shape=jax.ShapeDtypeStruct((N, 5), f32),
    )(h, head_fc1_W, head_fc1_b, head_fc2_W, head_fc2_b, head_ln_g,
      head_ln_b, head_proj_W, head_proj_b, comp_fc1_W,
      comp_fc1_b.reshape(1, HID // 2), comp_fc2_W,
      comp_fc2_b.reshape(1, 1), comp_ln_g.reshape(1, 1),
      comp_ln_b.reshape(1, 1), comp_proj_W, comp_proj_b.reshape(1, 1))
    return out
